# Initial kernel scaffold; baseline (speedup 1.0000x reference)
import jax
import jax.numpy as jnp
from jax.experimental import pallas as pl


def kernel(positions, x_TNH, cache):
    raise NotImplementedError("write your pallas kernel here")


# speedup vs baseline: 1.7642x; 1.7642x over previous
import functools

import jax
import jax.numpy as jnp
from jax import lax
from jax.experimental import pallas as pl
from jax.experimental.pallas import tpu as pltpu
from jax.experimental.pallas import tpu_sc as plsc

_CHUNK = 128


def _make_gather(V2, T):
    info = plsc.get_sparse_core_info()
    NC, NS = info.num_cores, info.num_subcores
    NW = NC * NS
    b_per_w = T // NW
    n_chunks = b_per_w // _CHUNK
    mesh = plsc.VectorSubcoreMesh(core_axis_name="c", subcore_axis_name="s")

    @functools.partial(
        pl.kernel,
        mesh=mesh,
        out_type=jax.ShapeDtypeStruct((T, 128), jnp.float32),
        scratch_types=[
            pltpu.VMEM((n_chunks, _CHUNK), jnp.int32),
            pltpu.VMEM((2, _CHUNK, 128), jnp.float32),
            pltpu.SemaphoreType.DMA,
            pltpu.SemaphoreType.DMA,
        ],
    )
    def gather_k(pos_hbm, cache_hbm, out_hbm, idx_v, rows_v, sem0, sem1):
        wid = lax.axis_index("s") * NC + lax.axis_index("c")
        base = wid * b_per_w
        pltpu.sync_copy(pos_hbm.at[wid], idx_v)
        sems = (sem0, sem1)
        handles = [None, None]
        for j in range(n_chunks):
            b = j & 1
            if handles[b] is not None:
                handles[b].wait()
                pltpu.sync_copy(
                    rows_v.at[b],
                    out_hbm.at[pl.ds(base + (j - 2) * _CHUNK, _CHUNK)],
                )
            handles[b] = pltpu.async_copy(
                cache_hbm.at[idx_v.at[j]], rows_v.at[b], sems[b]
            )
        for j in range(n_chunks - 2, n_chunks):
            b = j & 1
            handles[b].wait()
            pltpu.sync_copy(
                rows_v.at[b],
                out_hbm.at[pl.ds(base + j * _CHUNK, _CHUNK)],
            )

    return gather_k


def _expand_mats():
    rows = lax.broadcasted_iota(jnp.int32, (32, 64), 0)
    cols = lax.broadcasted_iota(jnp.int32, (32, 64), 1)
    ec = (cols // 2 == rows).astype(jnp.float32)
    r64 = lax.broadcasted_iota(jnp.int32, (64, 64), 0)
    c64 = lax.broadcasted_iota(jnp.int32, (64, 64), 1)
    sign = jnp.where(r64 % 2 == 1, -1.0, 1.0)
    p = jnp.where(c64 == (r64 ^ 1), sign, 0.0).astype(jnp.float32)
    return ec, p


def _rot_body(cs2_ref, par_ref, x_ref, o_ref):
    cs2 = cs2_ref[...]
    par = par_ref[...]
    cs = cs2[:, :64] + par * (cs2[:, 64:] - cs2[:, :64])
    ec, p = _expand_mats()
    csx = jnp.dot(cs[:, :32], ec,
                  preferred_element_type=jnp.float32,
                  precision=lax.Precision.HIGHEST)
    snx = jnp.dot(cs[:, 32:], ec,
                  preferred_element_type=jnp.float32,
                  precision=lax.Precision.HIGHEST)
    x = x_ref[...]
    tb, n, h = x.shape
    rot = jnp.dot(x.reshape(tb * n, h), p,
                  preferred_element_type=jnp.float32,
                  precision=lax.Precision.DEFAULT).reshape(x.shape)
    o_ref[...] = x * csx[:, None, :] + rot * snx[:, None, :]


def kernel(positions, x_TNH, cache):
    T, N, H = x_TNH.shape
    V = cache.shape[0]
    NW = 32
    cache2 = cache.reshape(V // 2, 2 * H)
    pos_half = (positions >> 1).reshape(NW, T // (NW * _CHUNK), _CHUNK)
    par = (positions & 1).astype(jnp.float32).reshape(T, 1)

    cs2_TH = _make_gather(V // 2, T)(pos_half, cache2)

    TB = 512
    rotate = pl.pallas_call(
        _rot_body,
        grid=(T // TB,),
        in_specs=[
            pl.BlockSpec((TB, 2 * H), lambda i: (i, 0)),
            pl.BlockSpec((TB, 1), lambda i: (i, 0)),
            pl.BlockSpec((TB, N, H), lambda i: (i, 0, 0)),
        ],
        out_specs=pl.BlockSpec((TB, N, H), lambda i: (i, 0, 0)),
        out_shape=jax.ShapeDtypeStruct((T, N, H), jnp.float32),
    )
    return rotate(cs2_TH, par, x_TNH)


# speedup vs baseline: 5.1599x; 2.9248x over previous
import functools

import jax
import jax.numpy as jnp
from jax import lax
from jax.experimental import pallas as pl
from jax.experimental.pallas import tpu as pltpu
from jax.experimental.pallas import tpu_sc as plsc

_CHUNK = 128


def _make_gather(T):
    info = plsc.get_sparse_core_info()
    NC, NS = info.num_cores, info.num_subcores
    NW = NC * NS
    b_per_w = T // NW
    n_chunks = b_per_w // _CHUNK
    mesh = plsc.VectorSubcoreMesh(core_axis_name="c", subcore_axis_name="s")

    @functools.partial(
        pl.kernel,
        mesh=mesh,
        out_type=jax.ShapeDtypeStruct((T, 128), jnp.float32),
        scratch_types=[
            pltpu.VMEM((n_chunks, _CHUNK), jnp.int32),
            pltpu.VMEM((2, _CHUNK, 128), jnp.float32),
            pltpu.SemaphoreType.DMA,
            pltpu.SemaphoreType.DMA,
        ],
    )
    def gather_k(pos_hbm, cache_hbm, out_hbm, idx_v, rows_v, sem0, sem1):
        wid = lax.axis_index("s") * NC + lax.axis_index("c")
        base = wid * b_per_w
        pltpu.sync_copy(pos_hbm.at[wid], idx_v)
        sems = (sem0, sem1)
        handles = [None, None]
        for j in range(n_chunks):
            b = j & 1
            if handles[b] is not None:
                handles[b].wait()
                pltpu.sync_copy(
                    rows_v.at[b],
                    out_hbm.at[pl.ds(base + (j - 2) * _CHUNK, _CHUNK)],
                )
            handles[b] = pltpu.async_copy(
                cache_hbm.at[idx_v.at[j]], rows_v.at[b], sems[b]
            )
        for j in range(n_chunks - 2, n_chunks):
            b = j & 1
            handles[b].wait()
            pltpu.sync_copy(
                rows_v.at[b],
                out_hbm.at[pl.ds(base + j * _CHUNK, _CHUNK)],
            )

    return gather_k


def _expand_mats():
    r = lax.broadcasted_iota(jnp.int32, (64, 32), 0)
    c = lax.broadcasted_iota(jnp.int32, (64, 32), 1)
    ecl = (r // 2 == c).astype(jnp.float32)
    r64 = lax.broadcasted_iota(jnp.int32, (64, 64), 0)
    c64 = lax.broadcasted_iota(jnp.int32, (64, 64), 1)
    sign = jnp.where(r64 % 2 == 0, -1.0, 1.0)
    pl_mat = jnp.where(c64 == (r64 ^ 1), sign, 0.0).astype(jnp.float32)
    return ecl, pl_mat


def _rot_body(cs2_ref, par_ref, x_ref, o_ref):
    cs2 = cs2_ref[...]
    cst = jnp.transpose(cs2)
    par = par_ref[...]
    cs = cst[:64] + par * (cst[64:] - cst[:64])
    ecl, pl_mat = _expand_mats()
    csx = jnp.dot(ecl, cs[:32],
                  preferred_element_type=jnp.float32,
                  precision=lax.Precision.HIGHEST)
    snx = jnp.dot(ecl, cs[32:],
                  preferred_element_type=jnp.float32,
                  precision=lax.Precision.HIGHEST)
    n = x_ref.shape[0]
    for i in range(n):
        xi = x_ref[i]
        rot = jnp.dot(pl_mat, xi,
                      preferred_element_type=jnp.float32,
                      precision=lax.Precision.DEFAULT)
        o_ref[i] = xi * csx + rot * snx


def kernel(positions, x_TNH, cache):
    T, N, H = x_TNH.shape
    V = cache.shape[0]
    NW = 32
    cache2 = cache.reshape(V // 2, 2 * H)
    pos_half = (positions >> 1).reshape(NW, T // (NW * _CHUNK), _CHUNK)
    par = (positions & 1).astype(jnp.float32).reshape(1, T)
    x_t = jnp.transpose(x_TNH, (1, 2, 0))

    cs2_TH = _make_gather(T)(pos_half, cache2)

    TB = 512
    rotate = pl.pallas_call(
        _rot_body,
        grid=(T // TB,),
        in_specs=[
            pl.BlockSpec((TB, 2 * H), lambda i: (i, 0)),
            pl.BlockSpec((1, TB), lambda i: (0, i)),
            pl.BlockSpec((N, H, TB), lambda i: (0, 0, i)),
        ],
        out_specs=pl.BlockSpec((N, H, TB), lambda i: (0, 0, i)),
        out_shape=jax.ShapeDtypeStruct((N, H, T), jnp.float32),
    )
    out_t = rotate(cs2_TH, par, x_t)
    return jnp.transpose(out_t, (2, 0, 1))


# speedup vs baseline: 5.5120x; 1.0682x over previous
import functools

import jax
import jax.numpy as jnp
from jax import lax
from jax.experimental import pallas as pl
from jax.experimental.pallas import tpu as pltpu
from jax.experimental.pallas import tpu_sc as plsc

_CHUNK = 128


def _prep_body(ct_ref, o_ref):
    ct = ct_ref[...]
    rows = jnp.transpose(ct)
    o_ref[...] = jnp.concatenate(
        [rows, jnp.zeros_like(rows)], axis=1)


def _make_gather(V, T):
    info = plsc.get_sparse_core_info()
    NC, NS = info.num_cores, info.num_subcores
    NW = NC * NS
    b_per_w = T // NW
    n_chunks = b_per_w // _CHUNK
    mesh = plsc.VectorSubcoreMesh(core_axis_name="c", subcore_axis_name="s")

    @functools.partial(
        pl.kernel,
        mesh=mesh,
        out_type=jax.ShapeDtypeStruct((T, 128), jnp.float32),
        scratch_types=[
            pltpu.VMEM((n_chunks, _CHUNK), jnp.int32),
            pltpu.VMEM((2, _CHUNK, 128), jnp.float32),
            pltpu.SemaphoreType.DMA,
            pltpu.SemaphoreType.DMA,
        ],
    )
    def gather_k(pos_hbm, src_hbm, out_hbm, idx_v, rows_v, sem0, sem1):
        wid = lax.axis_index("s") * NC + lax.axis_index("c")
        base = wid * b_per_w
        pltpu.sync_copy(pos_hbm.at[wid], idx_v)
        sems = (sem0, sem1)
        handles = [None, None]
        for j in range(n_chunks):
            b = j & 1
            if handles[b] is not None:
                handles[b].wait()
                pltpu.sync_copy(
                    rows_v.at[b],
                    out_hbm.at[pl.ds(base + (j - 2) * _CHUNK, _CHUNK)],
                )
            handles[b] = pltpu.async_copy(
                src_hbm.at[idx_v.at[j]], rows_v.at[b], sems[b]
            )
        for j in range(n_chunks - 2, n_chunks):
            b = j & 1
            handles[b].wait()
            pltpu.sync_copy(
                rows_v.at[b],
                out_hbm.at[pl.ds(base + j * _CHUNK, _CHUNK)],
            )

    return gather_k


def _expand_mats():
    r = lax.broadcasted_iota(jnp.int32, (64, 32), 0)
    c = lax.broadcasted_iota(jnp.int32, (64, 32), 1)
    ecl = (r // 2 == c).astype(jnp.float32)
    r64 = lax.broadcasted_iota(jnp.int32, (64, 64), 0)
    c64 = lax.broadcasted_iota(jnp.int32, (64, 64), 1)
    sign = jnp.where(r64 % 2 == 0, -1.0, 1.0)
    pl_mat = jnp.where(c64 == (r64 ^ 1), sign, 0.0).astype(jnp.float32)
    return ecl, pl_mat


def _rot_body(cs2_ref, x_ref, o_ref):
    cs2 = cs2_ref[...]
    cst = jnp.transpose(cs2)
    cs = cst[:64]
    ecl, pl_mat = _expand_mats()
    csx = jnp.dot(ecl, cs[:32],
                  preferred_element_type=jnp.float32,
                  precision=lax.Precision.HIGHEST)
    snx = jnp.dot(ecl, cs[32:],
                  preferred_element_type=jnp.float32,
                  precision=lax.Precision.HIGHEST)
    n = x_ref.shape[0]
    for i in range(n):
        xi = x_ref[i]
        rot = jnp.dot(pl_mat, xi,
                      preferred_element_type=jnp.float32,
                      precision=lax.Precision.DEFAULT)
        o_ref[i] = xi * csx + rot * snx


def kernel(positions, x_TNH, cache):
    T, N, H = x_TNH.shape
    V = cache.shape[0]
    NW = 32
    cache_t = jnp.transpose(cache)
    pos_idx = positions.reshape(NW, T // (NW * _CHUNK), _CHUNK)
    x_t = jnp.transpose(x_TNH, (1, 2, 0))

    TBC = 2048
    prep = pl.pallas_call(
        _prep_body,
        grid=(V // TBC,),
        in_specs=[pl.BlockSpec((H, TBC), lambda i: (0, i))],
        out_specs=pl.BlockSpec((TBC, 2 * H), lambda i: (i, 0)),
        out_shape=jax.ShapeDtypeStruct((V, 2 * H), jnp.float32),
    )
    src = prep(cache_t)

    cs2_TH = _make_gather(V, T)(pos_idx, src)

    TB = 512
    rotate = pl.pallas_call(
        _rot_body,
        grid=(T // TB,),
        in_specs=[
            pl.BlockSpec((TB, 2 * H), lambda i: (i, 0)),
            pl.BlockSpec((N, H, TB), lambda i: (0, 0, i)),
        ],
        out_specs=pl.BlockSpec((N, H, TB), lambda i: (0, 0, i)),
        out_shape=jax.ShapeDtypeStruct((N, H, T), jnp.float32),
    )
    out_t = rotate(cs2_TH, x_t)
    return jnp.transpose(out_t, (2, 0, 1))


# speedup vs baseline: 6.9913x; 1.2684x over previous
import functools

import jax
import jax.numpy as jnp
from jax import lax
from jax.experimental import pallas as pl
from jax.experimental.pallas import tpu as pltpu
from jax.experimental.pallas import tpu_sc as plsc

_CHUNK = 128


def _prep_body(ct_ref, o_ref):
    ct = ct_ref[...]
    rows = jnp.transpose(ct)
    o_ref[...] = jnp.concatenate(
        [rows, jnp.zeros_like(rows)], axis=1)


def _make_gather(V, T):
    info = plsc.get_sparse_core_info()
    NC, NS = info.num_cores, info.num_subcores
    NW = NC * NS
    b_per_w = T // NW
    n_chunks = b_per_w // _CHUNK
    mesh = plsc.VectorSubcoreMesh(core_axis_name="c", subcore_axis_name="s")

    @functools.partial(
        pl.kernel,
        mesh=mesh,
        out_type=jax.ShapeDtypeStruct((T, 128), jnp.float32),
        scratch_types=[
            pltpu.VMEM((n_chunks, _CHUNK), jnp.int32),
            pltpu.VMEM((2, _CHUNK, 128), jnp.float32),
            pltpu.SemaphoreType.DMA,
            pltpu.SemaphoreType.DMA,
        ],
    )
    def gather_k(pos_hbm, src_hbm, out_hbm, idx_v, rows_v, sem0, sem1):
        wid = lax.axis_index("s") * NC + lax.axis_index("c")
        base = wid * b_per_w
        pltpu.sync_copy(pos_hbm.at[wid], idx_v)
        sems = (sem0, sem1)
        handles = [None, None]
        for j in range(n_chunks):
            b = j & 1
            if handles[b] is not None:
                handles[b].wait()
                pltpu.sync_copy(
                    rows_v.at[b],
                    out_hbm.at[pl.ds(base + (j - 2) * _CHUNK, _CHUNK)],
                )
            handles[b] = pltpu.async_copy(
                src_hbm.at[idx_v.at[j]], rows_v.at[b], sems[b]
            )
        for j in range(n_chunks - 2, n_chunks):
            b = j & 1
            handles[b].wait()
            pltpu.sync_copy(
                rows_v.at[b],
                out_hbm.at[pl.ds(base + j * _CHUNK, _CHUNK)],
            )

    return gather_k


def _expand_mats():
    r = lax.broadcasted_iota(jnp.int32, (64, 32), 0)
    c = lax.broadcasted_iota(jnp.int32, (64, 32), 1)
    ecl = (r // 2 == c).astype(jnp.float32)
    r64 = lax.broadcasted_iota(jnp.int32, (64, 64), 0)
    c64 = lax.broadcasted_iota(jnp.int32, (64, 64), 1)
    sign = jnp.where(r64 % 2 == 0, -1.0, 1.0)
    pl_mat = jnp.where(c64 == (r64 ^ 1), sign, 0.0).astype(jnp.float32)
    return ecl, pl_mat


def _rot_body(cs2_ref, x_ref, o_ref):
    cs2 = cs2_ref[...]
    cst = jnp.transpose(cs2)
    cs = cst[:64]
    ecl, pl_mat = _expand_mats()
    csx = jnp.dot(ecl, cs[:32],
                  preferred_element_type=jnp.float32,
                  precision=lax.Precision.HIGHEST)
    snx = jnp.dot(ecl, cs[32:],
                  preferred_element_type=jnp.float32,
                  precision=lax.Precision.HIGHEST)
    n = x_ref.shape[0]
    for i in range(n):
        xi = x_ref[i]
        rot = jnp.dot(pl_mat, xi,
                      preferred_element_type=jnp.float32,
                      precision=lax.Precision.DEFAULT)
        o_ref[i] = xi * csx + rot * snx


def kernel(positions, x_TNH, cache):
    T, N, H = x_TNH.shape
    V = cache.shape[0]
    NW = 32
    cache_t = jnp.transpose(cache)
    pos_idx = positions.reshape(NW, T // (NW * _CHUNK), _CHUNK)
    x_t = jnp.transpose(x_TNH, (1, 2, 0))

    TBC = 8192
    prep = pl.pallas_call(
        _prep_body,
        grid=(V // TBC,),
        in_specs=[pl.BlockSpec((H, TBC), lambda i: (0, i))],
        out_specs=pl.BlockSpec((TBC, 2 * H), lambda i: (i, 0)),
        out_shape=jax.ShapeDtypeStruct((V, 2 * H), jnp.float32),
    )
    src = prep(cache_t)

    cs2_TH = _make_gather(V, T)(pos_idx, src)

    TB = 1024
    rotate = pl.pallas_call(
        _rot_body,
        grid=(T // TB,),
        in_specs=[
            pl.BlockSpec((TB, 2 * H), lambda i: (i, 0)),
            pl.BlockSpec((N, H, TB), lambda i: (0, 0, i)),
        ],
        out_specs=pl.BlockSpec((N, H, TB), lambda i: (0, 0, i)),
        out_shape=jax.ShapeDtypeStruct((N, H, T), jnp.float32),
    )
    out_t = rotate(cs2_TH, x_t)
    return jnp.transpose(out_t, (2, 0, 1))


# speedup vs baseline: 7.0278x; 1.0052x over previous
import functools

import jax
import jax.numpy as jnp
from jax import lax
from jax.experimental import pallas as pl
from jax.experimental.pallas import tpu as pltpu
from jax.experimental.pallas import tpu_sc as plsc

_CHUNK = 128


def _prep_body(ct_ref, o_ref):
    ct = ct_ref[...]
    rows = jnp.transpose(ct)
    o_ref[...] = jnp.concatenate(
        [rows, jnp.zeros_like(rows)], axis=1)


def _make_gather(V, T):
    info = plsc.get_sparse_core_info()
    NC, NS = info.num_cores, info.num_subcores
    NW = NC * NS
    b_per_w = T // NW
    n_chunks = b_per_w // _CHUNK
    mesh = plsc.VectorSubcoreMesh(core_axis_name="c", subcore_axis_name="s")

    @functools.partial(
        pl.kernel,
        mesh=mesh,
        out_type=jax.ShapeDtypeStruct((T, 128), jnp.float32),
        scratch_types=[
            pltpu.VMEM((n_chunks, _CHUNK), jnp.int32),
            pltpu.VMEM((4, _CHUNK, 128), jnp.float32),
            pltpu.SemaphoreType.DMA,
            pltpu.SemaphoreType.DMA,
            pltpu.SemaphoreType.DMA,
            pltpu.SemaphoreType.DMA,
        ],
    )
    def gather_k(pos_hbm, src_hbm, out_hbm, idx_v, rows_v, s0, s1, s2, s3):
        wid = lax.axis_index("s") * NC + lax.axis_index("c")
        base = wid * b_per_w
        pltpu.sync_copy(pos_hbm.at[wid], idx_v)
        sems = (s0, s1, s2, s3)
        nb = 4
        handles = [None] * nb
        for j in range(n_chunks):
            b = j % nb
            if handles[b] is not None:
                handles[b].wait()
                pltpu.sync_copy(
                    rows_v.at[b],
                    out_hbm.at[pl.ds(base + (j - nb) * _CHUNK, _CHUNK)],
                )
            handles[b] = pltpu.async_copy(
                src_hbm.at[idx_v.at[j]], rows_v.at[b], sems[b]
            )
        for j in range(max(0, n_chunks - nb), n_chunks):
            b = j % nb
            handles[b].wait()
            pltpu.sync_copy(
                rows_v.at[b],
                out_hbm.at[pl.ds(base + j * _CHUNK, _CHUNK)],
            )

    return gather_k


def _expand_mats():
    r = lax.broadcasted_iota(jnp.int32, (64, 32), 0)
    c = lax.broadcasted_iota(jnp.int32, (64, 32), 1)
    ecl = (r // 2 == c).astype(jnp.float32)
    r64 = lax.broadcasted_iota(jnp.int32, (64, 64), 0)
    c64 = lax.broadcasted_iota(jnp.int32, (64, 64), 1)
    sign = jnp.where(r64 % 2 == 0, -1.0, 1.0)
    pl_mat = jnp.where(c64 == (r64 ^ 1), sign, 0.0).astype(jnp.float32)
    return ecl, pl_mat


def _rot_body(cs2_ref, x_ref, o_ref):
    cs2 = cs2_ref[...]
    cst = jnp.transpose(cs2)
    cs = cst[:64]
    ecl, pl_mat = _expand_mats()
    csx = jnp.dot(ecl, cs[:32],
                  preferred_element_type=jnp.float32,
                  precision=lax.Precision.HIGHEST)
    snx = jnp.dot(ecl, cs[32:],
                  preferred_element_type=jnp.float32,
                  precision=lax.Precision.HIGHEST)
    n = x_ref.shape[0]
    for i in range(n):
        xi = x_ref[i]
        rot = jnp.dot(pl_mat, xi,
                      preferred_element_type=jnp.float32,
                      precision=lax.Precision.DEFAULT)
        o_ref[i] = xi * csx + rot * snx


def kernel(positions, x_TNH, cache):
    T, N, H = x_TNH.shape
    V = cache.shape[0]
    NW = 32
    cache_t = jnp.transpose(cache)
    pos_idx = positions.reshape(NW, T // (NW * _CHUNK), _CHUNK)
    x_t = jnp.transpose(x_TNH, (1, 2, 0))

    TBC = 8192
    prep = pl.pallas_call(
        _prep_body,
        grid=(V // TBC,),
        in_specs=[pl.BlockSpec((H, TBC), lambda i: (0, i))],
        out_specs=pl.BlockSpec((TBC, 2 * H), lambda i: (i, 0)),
        out_shape=jax.ShapeDtypeStruct((V, 2 * H), jnp.float32),
    )
    src = prep(cache_t)

    cs2_TH = _make_gather(V, T)(pos_idx, src)

    TB = 1024
    rotate = pl.pallas_call(
        _rot_body,
        grid=(T // TB,),
        in_specs=[
            pl.BlockSpec((TB, 2 * H), lambda i: (i, 0)),
            pl.BlockSpec((N, H, TB), lambda i: (0, 0, i)),
        ],
        out_specs=pl.BlockSpec((N, H, TB), lambda i: (0, 0, i)),
        out_shape=jax.ShapeDtypeStruct((N, H, T), jnp.float32),
    )
    out_t = rotate(cs2_TH, x_t)
    return jnp.transpose(out_t, (2, 0, 1))


# speedup vs baseline: 7.3555x; 1.0466x over previous
import functools

import jax
import jax.numpy as jnp
from jax import lax
from jax.experimental import pallas as pl
from jax.experimental.pallas import tpu as pltpu
from jax.experimental.pallas import tpu_sc as plsc

_CHUNK = 128


def _prep_body(ct_ref, o_ref):
    ct = ct_ref[...]
    rows = jnp.transpose(ct)
    o_ref[...] = jnp.concatenate(
        [rows, jnp.zeros_like(rows)], axis=1)


def _make_gather(V, T):
    info = plsc.get_sparse_core_info()
    NC, NS = info.num_cores, info.num_subcores
    NW = NC * NS
    b_per_w = T // NW
    n_chunks = b_per_w // _CHUNK
    mesh = plsc.VectorSubcoreMesh(core_axis_name="c", subcore_axis_name="s")

    @functools.partial(
        pl.kernel,
        mesh=mesh,
        out_type=jax.ShapeDtypeStruct((T, 128), jnp.float32),
        scratch_types=[
            pltpu.VMEM((n_chunks, _CHUNK), jnp.int32),
            pltpu.VMEM((4, _CHUNK, 128), jnp.float32),
            pltpu.SemaphoreType.DMA,
            pltpu.SemaphoreType.DMA,
            pltpu.SemaphoreType.DMA,
            pltpu.SemaphoreType.DMA,
        ],
    )
    def gather_k(pos_hbm, src_hbm, out_hbm, idx_v, rows_v, s0, s1, s2, s3):
        wid = lax.axis_index("s") * NC + lax.axis_index("c")
        base = wid * b_per_w
        pltpu.sync_copy(pos_hbm.at[wid], idx_v)
        sems = (s0, s1, s2, s3)
        nb = 4
        handles = [None] * nb
        for j in range(n_chunks):
            b = j % nb
            if handles[b] is not None:
                handles[b].wait()
                pltpu.sync_copy(
                    rows_v.at[b],
                    out_hbm.at[pl.ds(base + (j - nb) * _CHUNK, _CHUNK)],
                )
            handles[b] = pltpu.async_copy(
                src_hbm.at[idx_v.at[j]], rows_v.at[b], sems[b]
            )
        for j in range(max(0, n_chunks - nb), n_chunks):
            b = j % nb
            handles[b].wait()
            pltpu.sync_copy(
                rows_v.at[b],
                out_hbm.at[pl.ds(base + j * _CHUNK, _CHUNK)],
            )

    return gather_k


def _expand_mats():
    r = lax.broadcasted_iota(jnp.int32, (64, 32), 0)
    c = lax.broadcasted_iota(jnp.int32, (64, 32), 1)
    ecl = (r // 2 == c).astype(jnp.float32)
    r64 = lax.broadcasted_iota(jnp.int32, (64, 64), 0)
    c64 = lax.broadcasted_iota(jnp.int32, (64, 64), 1)
    sign = jnp.where(r64 % 2 == 0, -1.0, 1.0)
    pl_mat = jnp.where(c64 == (r64 ^ 1), sign, 0.0).astype(jnp.float32)
    return ecl, pl_mat


def _rot_body(cs2_ref, x_ref, o_ref):
    cs2 = cs2_ref[...]
    cst = jnp.transpose(cs2)
    cs = cst[:64]
    ecl, pl_mat = _expand_mats()
    csx = jnp.dot(ecl, cs[:32],
                  preferred_element_type=jnp.float32,
                  precision=lax.Precision.HIGHEST)
    snx = jnp.dot(ecl, cs[32:],
                  preferred_element_type=jnp.float32,
                  precision=lax.Precision.HIGHEST)
    n = x_ref.shape[0]
    for i in range(n):
        xi = x_ref[i]
        rot = jnp.dot(pl_mat, xi,
                      preferred_element_type=jnp.float32,
                      precision=lax.Precision.DEFAULT)
        o_ref[i] = xi * csx + rot * snx


def kernel(positions, x_TNH, cache):
    T, N, H = x_TNH.shape
    V = cache.shape[0]
    NW = 32
    cache_t = jnp.transpose(cache)
    pos_idx = positions.reshape(NW, T // (NW * _CHUNK), _CHUNK)
    x_t = jnp.transpose(x_TNH, (1, 2, 0))

    TBC = 16384
    prep = pl.pallas_call(
        _prep_body,
        grid=(V // TBC,),
        in_specs=[pl.BlockSpec((H, TBC), lambda i: (0, i))],
        out_specs=pl.BlockSpec((TBC, 2 * H), lambda i: (i, 0)),
        out_shape=jax.ShapeDtypeStruct((V, 2 * H), jnp.float32),
    )
    src = prep(cache_t)

    cs2_TH = _make_gather(V, T)(pos_idx, src)

    TB = 2048
    rotate = pl.pallas_call(
        _rot_body,
        grid=(T // TB,),
        in_specs=[
            pl.BlockSpec((TB, 2 * H), lambda i: (i, 0)),
            pl.BlockSpec((N, H, TB), lambda i: (0, 0, i)),
        ],
        out_specs=pl.BlockSpec((N, H, TB), lambda i: (0, 0, i)),
        out_shape=jax.ShapeDtypeStruct((N, H, T), jnp.float32),
    )
    out_t = rotate(cs2_TH, x_t)
    return jnp.transpose(out_t, (2, 0, 1))
